# Initial kernel scaffold; baseline (speedup 1.0000x reference)
import jax
import jax.numpy as jnp
from jax.experimental import pallas as pl


def kernel(features, points, neighbors, unary1_weight, kpconv_weights, kpconv_kernel_points, unary2_weight, unary_shortcut_weight):
    raise NotImplementedError("write your pallas kernel here")


# speedup vs baseline: 1.1212x; 1.1212x over previous
import jax
import jax.numpy as jnp
from jax import lax
from jax.experimental import pallas as pl
from jax.experimental.pallas import tpu as pltpu

_EPS = 1e-5
_SLOPE = 0.1
_KP_EXTENT = 1.2


def _lrelu(y):
    return jnp.where(y >= 0.0, y, _SLOPE * y)


def _gemm_stats_kernel(x_ref, w_ref, ps_ref, pq_ref):
    y = jnp.dot(x_ref[...], w_ref[...], preferred_element_type=jnp.float32)
    ps_ref[0, 0, :] = jnp.sum(y, axis=0)
    pq_ref[0, 0, :] = jnp.sum(y * y, axis=0)


def _gemm_stats(x, w, tn):
    n, kdim = x.shape
    c = w.shape[1]
    t = n // tn
    return pl.pallas_call(
        _gemm_stats_kernel,
        out_shape=(jax.ShapeDtypeStruct((t, 1, c), jnp.float32),
                   jax.ShapeDtypeStruct((t, 1, c), jnp.float32)),
        grid=(t,),
        in_specs=[pl.BlockSpec((tn, kdim), lambda i: (i, 0)),
                  pl.BlockSpec((kdim, c), lambda i: (0, 0))],
        out_specs=(pl.BlockSpec((1, 1, c), lambda i: (i, 0, 0)),
                   pl.BlockSpec((1, 1, c), lambda i: (i, 0, 0))),
        compiler_params=pltpu.CompilerParams(
            dimension_semantics=("parallel",)),
    )(x, w)


def _scale_shift(ps, pq, n):
    s = jnp.sum(ps[:, 0, :], axis=0)
    q = jnp.sum(pq[:, 0, :], axis=0)
    mean = s / n
    var = q / n - mean * mean
    inv = lax.rsqrt(var + _EPS)
    return inv[None, :], (-mean * inv)[None, :]


def _gemm_affine_kernel(x_ref, w_ref, a_ref, b_ref, o_ref):
    y = jnp.dot(x_ref[...], w_ref[...], preferred_element_type=jnp.float32)
    o_ref[...] = _lrelu(y * a_ref[...] + b_ref[...])


def _gemm_affine(x, w, a, b, tn):
    n, kdim = x.shape
    c = w.shape[1]
    t = n // tn
    return pl.pallas_call(
        _gemm_affine_kernel,
        out_shape=jax.ShapeDtypeStruct((n, c), jnp.float32),
        grid=(t,),
        in_specs=[pl.BlockSpec((tn, kdim), lambda i: (i, 0)),
                  pl.BlockSpec((kdim, c), lambda i: (0, 0)),
                  pl.BlockSpec((1, c), lambda i: (0, 0)),
                  pl.BlockSpec((1, c), lambda i: (0, 0))],
        out_specs=pl.BlockSpec((tn, c), lambda i: (i, 0)),
        compiler_params=pltpu.CompilerParams(
            dimension_semantics=("parallel",)),
    )(x, w, a, b)


def _conv_gemm_kernel(wt_ref, w_ref, inv_ref, o_ref, ps_ref, pq_ref):
    y = jnp.dot(wt_ref[...], w_ref[...], preferred_element_type=jnp.float32)
    y = y * inv_ref[...]
    o_ref[...] = y
    ps_ref[0, 0, :] = jnp.sum(y, axis=0)
    pq_ref[0, 0, :] = jnp.sum(y * y, axis=0)


def _conv_gemm(weighted, w, inv, tn):
    n, kdim = weighted.shape
    c = w.shape[1]
    t = n // tn
    return pl.pallas_call(
        _conv_gemm_kernel,
        out_shape=(jax.ShapeDtypeStruct((n, c), jnp.float32),
                   jax.ShapeDtypeStruct((t, 1, c), jnp.float32),
                   jax.ShapeDtypeStruct((t, 1, c), jnp.float32)),
        grid=(t,),
        in_specs=[pl.BlockSpec((tn, kdim), lambda i: (i, 0)),
                  pl.BlockSpec((kdim, c), lambda i: (0, 0)),
                  pl.BlockSpec((tn, 1), lambda i: (i, 0))],
        out_specs=(pl.BlockSpec((tn, c), lambda i: (i, 0)),
                   pl.BlockSpec((1, 1, c), lambda i: (i, 0, 0)),
                   pl.BlockSpec((1, 1, c), lambda i: (i, 0, 0))),
        compiler_params=pltpu.CompilerParams(
            dimension_semantics=("parallel",)),
    )(weighted, w, inv)


def _final_stats_kernel(cv_ref, ft_ref, a3_ref, b3_ref, w2_ref, ws_ref,
                        ps2_ref, pq2_ref, pss_ref, pqs_ref):
    x3 = _lrelu(cv_ref[...] * a3_ref[...] + b3_ref[...])
    y2 = jnp.dot(x3, w2_ref[...], preferred_element_type=jnp.float32)
    sc = jnp.dot(ft_ref[...], ws_ref[...], preferred_element_type=jnp.float32)
    ps2_ref[0, 0, :] = jnp.sum(y2, axis=0)
    pq2_ref[0, 0, :] = jnp.sum(y2 * y2, axis=0)
    pss_ref[0, 0, :] = jnp.sum(sc, axis=0)
    pqs_ref[0, 0, :] = jnp.sum(sc * sc, axis=0)


def _final_stats(conv, feat, a3, b3, w2, ws, tn):
    n, cm = conv.shape
    cin = feat.shape[1]
    c = w2.shape[1]
    t = n // tn
    stat = jax.ShapeDtypeStruct((t, 1, c), jnp.float32)
    return pl.pallas_call(
        _final_stats_kernel,
        out_shape=(stat, stat, stat, stat),
        grid=(t,),
        in_specs=[pl.BlockSpec((tn, cm), lambda i: (i, 0)),
                  pl.BlockSpec((tn, cin), lambda i: (i, 0)),
                  pl.BlockSpec((1, cm), lambda i: (0, 0)),
                  pl.BlockSpec((1, cm), lambda i: (0, 0)),
                  pl.BlockSpec((cm, c), lambda i: (0, 0)),
                  pl.BlockSpec((cin, c), lambda i: (0, 0))],
        out_specs=(pl.BlockSpec((1, 1, c), lambda i: (i, 0, 0)),) * 4,
        compiler_params=pltpu.CompilerParams(
            dimension_semantics=("parallel",)),
    )(conv, feat, a3, b3, w2, ws)


def _final_apply_kernel(cv_ref, ft_ref, a3_ref, b3_ref, w2_ref, ws_ref,
                        a2_ref, b2_ref, as_ref, bs_ref, o_ref):
    x3 = _lrelu(cv_ref[...] * a3_ref[...] + b3_ref[...])
    y2 = jnp.dot(x3, w2_ref[...], preferred_element_type=jnp.float32)
    sc = jnp.dot(ft_ref[...], ws_ref[...], preferred_element_type=jnp.float32)
    y = y2 * a2_ref[...] + b2_ref[...] + (sc * as_ref[...] + bs_ref[...])
    o_ref[...] = _lrelu(y)


def _final_apply(conv, feat, a3, b3, w2, ws, a2, b2, a_s, b_s, tn):
    n, cm = conv.shape
    cin = feat.shape[1]
    c = w2.shape[1]
    t = n // tn
    vec_c = pl.BlockSpec((1, c), lambda i: (0, 0))
    return pl.pallas_call(
        _final_apply_kernel,
        out_shape=jax.ShapeDtypeStruct((n, c), jnp.float32),
        grid=(t,),
        in_specs=[pl.BlockSpec((tn, cm), lambda i: (i, 0)),
                  pl.BlockSpec((tn, cin), lambda i: (i, 0)),
                  pl.BlockSpec((1, cm), lambda i: (0, 0)),
                  pl.BlockSpec((1, cm), lambda i: (0, 0)),
                  pl.BlockSpec((cm, c), lambda i: (0, 0)),
                  pl.BlockSpec((cin, c), lambda i: (0, 0)),
                  vec_c, vec_c, vec_c, vec_c],
        out_specs=pl.BlockSpec((tn, c), lambda i: (i, 0)),
        compiler_params=pltpu.CompilerParams(
            dimension_semantics=("parallel",)),
    )(conv, feat, a3, b3, w2, ws, a2, b2, a_s, b_s)


def kernel(features, points, neighbors, unary1_weight, kpconv_weights,
           kpconv_kernel_points, unary2_weight, unary_shortcut_weight):
    n = features.shape[0]
    kp = kpconv_weights.shape[0]
    cm = kpconv_weights.shape[1]

    ps, pq = _gemm_stats(features, unary1_weight, tn=4096)
    a1, b1 = _scale_shift(ps, pq, n)
    x1 = _gemm_affine(features, unary1_weight, a1, b1, tn=4096)

    s_pad = jnp.concatenate([points, jnp.zeros_like(points[:1, :]) + 1e6], 0)
    nb_pos = s_pad[neighbors] - points[:, None, :]
    diffs = nb_pos[:, :, None, :] - kpconv_kernel_points[None, None]
    sq_d = jnp.sum(diffs * diffs, axis=3)
    all_w = jnp.maximum(0.0, 1.0 - jnp.sqrt(sq_d) / _KP_EXTENT)
    all_w = jnp.swapaxes(all_w, 1, 2)

    x_pad = jnp.concatenate([x1, jnp.zeros_like(x1[:1, :])], 0)
    nb_x = x_pad[neighbors]
    weighted = jnp.matmul(all_w, nb_x).reshape(n, kp * cm)

    nb_sum = jnp.sum(nb_x, axis=-1)
    cnt = jnp.maximum(jnp.sum((nb_sum > 0.0).astype(jnp.int32), axis=-1), 1)
    inv = (1.0 / cnt.astype(jnp.float32)).reshape(n, 1)

    conv, ps3, pq3 = _conv_gemm(weighted,
                                kpconv_weights.reshape(kp * cm, cm), inv,
                                tn=1024)
    a3, b3 = _scale_shift(ps3, pq3, n)

    ps2, pq2, pss, pqs = _final_stats(conv, features, a3, b3, unary2_weight,
                                      unary_shortcut_weight, tn=4096)
    a2, b2 = _scale_shift(ps2, pq2, n)
    a_s, b_s = _scale_shift(pss, pqs, n)
    return _final_apply(conv, features, a3, b3, unary2_weight,
                        unary_shortcut_weight, a2, b2, a_s, b_s, tn=2048)
